# baseline (speedup 1.0000x reference)
import functools

import jax
import jax.numpy as jnp
from jax import lax
from jax.experimental import pallas as pl
from jax.experimental.pallas import tpu as pltpu
from jax.experimental.pallas import tpu_sc as plsc

N = 100000
E = 3200000
NC = 2
NS = 16
NW = NC * NS
EPAD = 3276800
GCHUNK = 1024
GITER = EPAD // NW // GCHUNK
GC_SLOW = 70
GC_FAST = 130
SC_SLOW = 88
SC_FAST = 112
NAGG = 100352

_mesh = plsc.VectorSubcoreMesh(core_axis_name="c", subcore_axis_name="s")
_sc_params = pltpu.CompilerParams(use_tc_tiling_on_sc=False)


@functools.partial(
    pl.kernel,
    out_type=(
        jax.ShapeDtypeStruct((EPAD, 16), jnp.float32),
        jax.ShapeDtypeStruct((EPAD, 16), jnp.float32),
    ),
    mesh=_mesh,
    scratch_types=[
        pltpu.VMEM((2, 8, 128), jnp.int32),
        pltpu.VMEM((2, 8, 128), jnp.int32),
        pltpu.VMEM((2, GCHUNK, 16), jnp.float32),
        pltpu.VMEM((2, GCHUNK, 16), jnp.float32),
        pltpu.SemaphoreType.DMA,
        pltpu.SemaphoreType.DMA,
        pltpu.SemaphoreType.DMA,
    ],
    compiler_params=_sc_params,
)
def _sc_gather(table_hbm, sidx_hbm, didx_hbm, srows_hbm, drows_hbm,
               idxs_v, idxd_v, bufs_v, bufd_v, gsem, wsem0, wsem1):
    c = lax.axis_index("c")
    s = lax.axis_index("s")
    wsems = (wsem0, wsem1)
    pert = jnp.where(c == 0, GC_SLOW, GC_FAST)
    start = jnp.where(c == 0, s * GC_SLOW, NS * GC_SLOW + s * GC_FAST)

    def outer(g, carry):
        for p in range(2):
            i = g * 2 + p
            base = pl.multiple_of((start + i) * GCHUNK, 1024)
            row0 = pl.multiple_of(base // 128, 8)

            @pl.when(g > 0)
            def _drain():
                pltpu.make_async_copy(
                    bufs_v.at[p], srows_hbm.at[pl.ds(base, GCHUNK)],
                    wsems[p]).wait()
                pltpu.make_async_copy(
                    bufd_v.at[p], drows_hbm.at[pl.ds(base, GCHUNK)],
                    wsems[p]).wait()

            pltpu.sync_copy(sidx_hbm.at[pl.ds(row0, 8)], idxs_v.at[p])
            pltpu.sync_copy(didx_hbm.at[pl.ds(row0, 8)], idxd_v.at[p])
            descs = []
            for j in range(8):
                descs.append(pltpu.async_copy(
                    table_hbm.at[idxs_v.at[p].at[j]],
                    bufs_v.at[p].at[pl.ds(j * 128, 128)], gsem))
                descs.append(pltpu.async_copy(
                    table_hbm.at[idxd_v.at[p].at[j]],
                    bufd_v.at[p].at[pl.ds(j * 128, 128)], gsem))
            for d in descs:
                d.wait()
            pltpu.async_copy(
                bufs_v.at[p], srows_hbm.at[pl.ds(base, GCHUNK)], wsems[p])
            pltpu.async_copy(
                bufd_v.at[p], drows_hbm.at[pl.ds(base, GCHUNK)], wsems[p])
        return carry

    lax.fori_loop(0, pert // 2, outer, 0)

    for p in range(2):
        base = pl.multiple_of((start + pert - 2 + p) * GCHUNK, 1024)
        pltpu.make_async_copy(
            bufs_v.at[p], srows_hbm.at[pl.ds(base, GCHUNK)], wsems[p]).wait()
        pltpu.make_async_copy(
            bufd_v.at[p], drows_hbm.at[pl.ds(base, GCHUNK)], wsems[p]).wait()


@functools.partial(
    pl.kernel,
    out_type=jax.ShapeDtypeStruct((NC, NAGG), jnp.float32),
    mesh=_mesh,
    scratch_types=[
        pltpu.VMEM((8, 128), jnp.int32),
        pltpu.VMEM((8, 128), jnp.float32),
        pltpu.VMEM((2048,), jnp.float32),
        pltpu.VMEM_SHARED((NAGG,), jnp.float32),
        pltpu.SemaphoreType.DMA,
    ],
    compiler_params=_sc_params,
)
def _sc_scatter(didx_hbm, m_hbm, agg_hbm, idx_v, val_v, zbuf_v, agg_sp, sem):
    c = lax.axis_index("c")
    s = lax.axis_index("s")

    @pl.when(s == 0)
    def _zero():
        def zb(k, carry):
            zbuf_v[pl.ds(k * 16, 16)] = jnp.zeros((16,), jnp.float32)
            return carry
        lax.fori_loop(0, 2048 // 16, zb, 0)

        def zs(k, carry):
            pltpu.sync_copy(zbuf_v, agg_sp.at[pl.ds(k * 2048, 2048)])
            return carry
        lax.fori_loop(0, NAGG // 2048, zs, 0)

    plsc.subcore_barrier()

    pert = jnp.where(c == 0, SC_SLOW, SC_FAST)
    start = jnp.where(c == 0, s * SC_SLOW, NS * SC_SLOW + s * SC_FAST)

    def chunk(i, carry):
        row0 = pl.multiple_of((start + i) * 8, 8)
        pltpu.sync_copy(didx_hbm.at[pl.ds(row0, 8)], idx_v)
        pltpu.sync_copy(m_hbm.at[pl.ds(row0, 8)], val_v)
        descs = []
        for j in range(8):
            descs.append(pltpu.async_copy(
                val_v.at[j], agg_sp.at[idx_v.at[j]], sem, add=True))
        for d in descs:
            d.wait()
        return carry

    lax.fori_loop(0, pert, chunk, 0)

    plsc.subcore_barrier()

    @pl.when(s == 0)
    def _writeout():
        pltpu.sync_copy(agg_sp, agg_hbm.at[c])


def _phi_body(s_ref, d_ref, k0s_ref, k0d_ref, b0_ref, k1_ref, b1_ref,
              k2_ref, b2_ref, m_ref):
    h = jnp.tanh(
        jnp.dot(s_ref[...], k0s_ref[...], preferred_element_type=jnp.float32)
        + jnp.dot(d_ref[...], k0d_ref[...], preferred_element_type=jnp.float32)
        + b0_ref[...])
    h = jnp.tanh(
        jnp.dot(h, k1_ref[...], preferred_element_type=jnp.float32)
        + b1_ref[...])
    m_ref[...] = (
        jnp.dot(h, k2_ref[...], preferred_element_type=jnp.float32)
        + b2_ref[...])


def _run_phi(sp, dp, k0s, k0d, b0, k1, b1, k2, b2):
    rows = EPAD // 8
    blk = 8192
    grid = rows // blk
    full = lambda shape: pl.BlockSpec(shape, lambda i: (0, 0))
    return pl.pallas_call(
        _phi_body,
        grid=(grid,),
        in_specs=[
            pl.BlockSpec((blk, 128), lambda i: (i, 0)),
            pl.BlockSpec((blk, 128), lambda i: (i, 0)),
            full((128, 256)), full((128, 256)), full((1, 256)),
            full((256, 256)), full((1, 256)),
            full((256, 8)), full((1, 8)),
        ],
        out_specs=pl.BlockSpec((blk, 8), lambda i: (i, 0)),
        out_shape=jax.ShapeDtypeStruct((rows, 8), jnp.float32),
    )(sp, dp, k0s, k0d, b0, k1, b1, k2, b2)


def _gamma_body(x_ref, a0_ref, a1_ref, xl_ref, g0x_ref, g0a_ref, b0_ref,
                g1_ref, b1_ref, g2_ref, b2_ref, o_ref):
    a = a0_ref[...] + a1_ref[...]
    h = jnp.tanh(
        jnp.dot(x_ref[...], g0x_ref[...], preferred_element_type=jnp.float32)
        + jnp.dot(a, g0a_ref[...], preferred_element_type=jnp.float32)
        + b0_ref[...])
    h = jnp.tanh(
        jnp.dot(h, g1_ref[...], preferred_element_type=jnp.float32)
        + b1_ref[...])
    o_ref[...] = (xl_ref[...]
                  + jnp.dot(h, g2_ref[...], preferred_element_type=jnp.float32)
                  + b2_ref[...])


def _run_gamma(xp, a0, a1, xl, g0x, g0a, b0, g1, b1, g2, b2):
    rows = N // 16
    return pl.pallas_call(
        _gamma_body,
        out_shape=jax.ShapeDtypeStruct((rows, 16), jnp.float32),
    )(xp, a0, a1, xl, g0x, g0a, b0, g1, b1, g2, b2)


def kernel(x, pos, edge_index, pW0, pb0, pW1, pb1, pW2, pb2,
           gW0, gb0, gW1, gb1, gW2, gb2):
    f32 = jnp.float32
    table = jnp.concatenate(
        [x, pos, jnp.zeros((N, 6), f32)], axis=1)

    src = edge_index[0]
    dst = edge_index[1]
    pad = EPAD - E
    src_p = jnp.concatenate([src, jnp.zeros((pad,), jnp.int32)])
    dst_p = jnp.concatenate([dst, jnp.full((pad,), N, jnp.int32)])
    sidx = src_p.reshape(EPAD // 128, 128)
    didx = dst_p.reshape(EPAD // 128, 128)

    srows, drows = _sc_gather(table, sidx, didx)

    i8 = jnp.eye(8, dtype=f32)
    ws = jnp.zeros((16, 32), f32).at[0:8].set(pW0[8:16]).at[8:10].set(pW0[16:18])
    wd = jnp.zeros((16, 32), f32).at[0:8].set(pW0[0:8]).at[8:10].set(-pW0[16:18])
    k0s = jnp.kron(i8, ws)
    k0d = jnp.kron(i8, wd)
    b0 = jnp.tile(pb0, 8).reshape(1, 256)
    k1 = jnp.kron(i8, pW1)
    b1 = jnp.tile(pb1, 8).reshape(1, 256)
    k2 = jnp.kron(i8, pW2)
    b2 = jnp.tile(pb2, 8).reshape(1, 8)

    sp = srows.reshape(EPAD // 8, 128)
    dp = drows.reshape(EPAD // 8, 128)
    m2d = _run_phi(sp, dp, k0s, k0d, b0, k1, b1, k2, b2)

    mrows = m2d.reshape(EPAD // 128, 128)
    agg2 = _sc_scatter(didx, mrows)

    i16 = jnp.eye(16, dtype=f32)
    g0x = jnp.kron(i16, gW0[0:8])
    g0a = jnp.kron(i16, gW0[8:9])
    gb0t = jnp.tile(gb0, 16).reshape(1, 512)
    g1 = jnp.kron(i16, gW1)
    gb1t = jnp.tile(gb1, 16).reshape(1, 512)
    g2 = jnp.kron(i16, gW2)
    gb2t = jnp.tile(gb2, 16).reshape(1, 16)

    xp = x.reshape(N // 16, 128)
    a0 = agg2[0, :N].reshape(N // 16, 16)
    a1 = agg2[1, :N].reshape(N // 16, 16)
    xl = x[:, 7].reshape(N // 16, 16)

    out = _run_gamma(xp, a0, a1, xl, g0x, g0a, gb0t, g1, gb1t, g2, gb2t)
    return out.reshape(N, 1)


# speedup vs baseline: 1.0147x; 1.0147x over previous
import functools

import jax
import jax.numpy as jnp
from jax import lax
from jax.experimental import pallas as pl
from jax.experimental.pallas import tpu as pltpu
from jax.experimental.pallas import tpu_sc as plsc

N = 100000
E = 3200000
NC = 2
NS = 16
NW = NC * NS
EPAD = 3276800
GCHUNK = 1024
GITER = EPAD // NW // GCHUNK
GC_SLOW = 70
GC_FAST = 130
SC_SLOW = 88
SC_FAST = 112
NAGG = 100352

_mesh = plsc.VectorSubcoreMesh(core_axis_name="c", subcore_axis_name="s")
_sc_params = pltpu.CompilerParams(use_tc_tiling_on_sc=False)


@functools.partial(
    pl.kernel,
    out_type=(
        jax.ShapeDtypeStruct((EPAD, 16), jnp.float32),
        jax.ShapeDtypeStruct((EPAD, 16), jnp.float32),
    ),
    mesh=_mesh,
    scratch_types=[
        pltpu.VMEM((2, 8, 128), jnp.int32),
        pltpu.VMEM((2, 8, 128), jnp.int32),
        pltpu.VMEM((2, GCHUNK, 16), jnp.float32),
        pltpu.VMEM((2, GCHUNK, 16), jnp.float32),
        pltpu.SemaphoreType.DMA,
        pltpu.SemaphoreType.DMA,
        pltpu.SemaphoreType.DMA,
    ],
    compiler_params=_sc_params,
)
def _sc_gather(table_hbm, sidx_hbm, didx_hbm, srows_hbm, drows_hbm,
               idxs_v, idxd_v, bufs_v, bufd_v, gsem, wsem0, wsem1):
    c = lax.axis_index("c")
    s = lax.axis_index("s")
    wsems = (wsem0, wsem1)
    pert = jnp.where(c == 1, GC_SLOW, GC_FAST)
    start = jnp.where(c == 1, s * GC_SLOW, NS * GC_SLOW + s * GC_FAST)

    def outer(g, carry):
        for p in range(2):
            i = g * 2 + p
            base = pl.multiple_of((start + i) * GCHUNK, 1024)
            row0 = pl.multiple_of(base // 128, 8)

            @pl.when(g > 0)
            def _drain():
                pltpu.make_async_copy(
                    bufs_v.at[p], srows_hbm.at[pl.ds(base, GCHUNK)],
                    wsems[p]).wait()
                pltpu.make_async_copy(
                    bufd_v.at[p], drows_hbm.at[pl.ds(base, GCHUNK)],
                    wsems[p]).wait()

            pltpu.sync_copy(sidx_hbm.at[pl.ds(row0, 8)], idxs_v.at[p])
            pltpu.sync_copy(didx_hbm.at[pl.ds(row0, 8)], idxd_v.at[p])
            descs = []
            for j in range(8):
                descs.append(pltpu.async_copy(
                    table_hbm.at[idxs_v.at[p].at[j]],
                    bufs_v.at[p].at[pl.ds(j * 128, 128)], gsem))
                descs.append(pltpu.async_copy(
                    table_hbm.at[idxd_v.at[p].at[j]],
                    bufd_v.at[p].at[pl.ds(j * 128, 128)], gsem))
            for d in descs:
                d.wait()
            pltpu.async_copy(
                bufs_v.at[p], srows_hbm.at[pl.ds(base, GCHUNK)], wsems[p])
            pltpu.async_copy(
                bufd_v.at[p], drows_hbm.at[pl.ds(base, GCHUNK)], wsems[p])
        return carry

    lax.fori_loop(0, pert // 2, outer, 0)

    for p in range(2):
        base = pl.multiple_of((start + pert - 2 + p) * GCHUNK, 1024)
        pltpu.make_async_copy(
            bufs_v.at[p], srows_hbm.at[pl.ds(base, GCHUNK)], wsems[p]).wait()
        pltpu.make_async_copy(
            bufd_v.at[p], drows_hbm.at[pl.ds(base, GCHUNK)], wsems[p]).wait()


@functools.partial(
    pl.kernel,
    out_type=jax.ShapeDtypeStruct((NC, NAGG), jnp.float32),
    mesh=_mesh,
    scratch_types=[
        pltpu.VMEM((8, 128), jnp.int32),
        pltpu.VMEM((8, 128), jnp.float32),
        pltpu.VMEM((2048,), jnp.float32),
        pltpu.VMEM_SHARED((NAGG,), jnp.float32),
        pltpu.SemaphoreType.DMA,
    ],
    compiler_params=_sc_params,
)
def _sc_scatter(didx_hbm, m_hbm, agg_hbm, idx_v, val_v, zbuf_v, agg_sp, sem):
    c = lax.axis_index("c")
    s = lax.axis_index("s")

    @pl.when(s == 0)
    def _zero():
        def zb(k, carry):
            zbuf_v[pl.ds(k * 16, 16)] = jnp.zeros((16,), jnp.float32)
            return carry
        lax.fori_loop(0, 2048 // 16, zb, 0)

        def zs(k, carry):
            pltpu.sync_copy(zbuf_v, agg_sp.at[pl.ds(k * 2048, 2048)])
            return carry
        lax.fori_loop(0, NAGG // 2048, zs, 0)

    plsc.subcore_barrier()

    pert = jnp.where(c == 1, SC_SLOW, SC_FAST)
    start = jnp.where(c == 1, s * SC_SLOW, NS * SC_SLOW + s * SC_FAST)

    def chunk(i, carry):
        row0 = pl.multiple_of((start + i) * 8, 8)
        pltpu.sync_copy(didx_hbm.at[pl.ds(row0, 8)], idx_v)
        pltpu.sync_copy(m_hbm.at[pl.ds(row0, 8)], val_v)
        descs = []
        for j in range(8):
            descs.append(pltpu.async_copy(
                val_v.at[j], agg_sp.at[idx_v.at[j]], sem, add=True))
        for d in descs:
            d.wait()
        return carry

    lax.fori_loop(0, pert, chunk, 0)

    plsc.subcore_barrier()

    @pl.when(s == 0)
    def _writeout():
        pltpu.sync_copy(agg_sp, agg_hbm.at[c])


def _phi_body(s_ref, d_ref, k0s_ref, k0d_ref, b0_ref, k1_ref, b1_ref,
              k2_ref, b2_ref, m_ref):
    h = jnp.tanh(
        jnp.dot(s_ref[...], k0s_ref[...], preferred_element_type=jnp.float32)
        + jnp.dot(d_ref[...], k0d_ref[...], preferred_element_type=jnp.float32)
        + b0_ref[...])
    h = jnp.tanh(
        jnp.dot(h, k1_ref[...], preferred_element_type=jnp.float32)
        + b1_ref[...])
    m_ref[...] = (
        jnp.dot(h, k2_ref[...], preferred_element_type=jnp.float32)
        + b2_ref[...])


def _run_phi(sp, dp, k0s, k0d, b0, k1, b1, k2, b2):
    rows = EPAD // 8
    blk = 8192
    grid = rows // blk
    full = lambda shape: pl.BlockSpec(shape, lambda i: (0, 0))
    return pl.pallas_call(
        _phi_body,
        grid=(grid,),
        in_specs=[
            pl.BlockSpec((blk, 128), lambda i: (i, 0)),
            pl.BlockSpec((blk, 128), lambda i: (i, 0)),
            full((128, 256)), full((128, 256)), full((1, 256)),
            full((256, 256)), full((1, 256)),
            full((256, 8)), full((1, 8)),
        ],
        out_specs=pl.BlockSpec((blk, 8), lambda i: (i, 0)),
        out_shape=jax.ShapeDtypeStruct((rows, 8), jnp.float32),
    )(sp, dp, k0s, k0d, b0, k1, b1, k2, b2)


def _gamma_body(x_ref, a0_ref, a1_ref, xl_ref, g0x_ref, g0a_ref, b0_ref,
                g1_ref, b1_ref, g2_ref, b2_ref, o_ref):
    a = a0_ref[...] + a1_ref[...]
    h = jnp.tanh(
        jnp.dot(x_ref[...], g0x_ref[...], preferred_element_type=jnp.float32)
        + jnp.dot(a, g0a_ref[...], preferred_element_type=jnp.float32)
        + b0_ref[...])
    h = jnp.tanh(
        jnp.dot(h, g1_ref[...], preferred_element_type=jnp.float32)
        + b1_ref[...])
    o_ref[...] = (xl_ref[...]
                  + jnp.dot(h, g2_ref[...], preferred_element_type=jnp.float32)
                  + b2_ref[...])


def _run_gamma(xp, a0, a1, xl, g0x, g0a, b0, g1, b1, g2, b2):
    rows = N // 16
    return pl.pallas_call(
        _gamma_body,
        out_shape=jax.ShapeDtypeStruct((rows, 16), jnp.float32),
    )(xp, a0, a1, xl, g0x, g0a, b0, g1, b1, g2, b2)


def kernel(x, pos, edge_index, pW0, pb0, pW1, pb1, pW2, pb2,
           gW0, gb0, gW1, gb1, gW2, gb2):
    f32 = jnp.float32
    table = jnp.concatenate(
        [x, pos, jnp.zeros((N, 6), f32)], axis=1)

    src = edge_index[0]
    dst = edge_index[1]
    pad = EPAD - E
    src_p = jnp.concatenate([src, jnp.zeros((pad,), jnp.int32)])
    dst_p = jnp.concatenate([dst, jnp.full((pad,), N, jnp.int32)])
    sidx = src_p.reshape(EPAD // 128, 128)
    didx = dst_p.reshape(EPAD // 128, 128)

    srows, drows = _sc_gather(table, sidx, didx)

    i8 = jnp.eye(8, dtype=f32)
    ws = jnp.zeros((16, 32), f32).at[0:8].set(pW0[8:16]).at[8:10].set(pW0[16:18])
    wd = jnp.zeros((16, 32), f32).at[0:8].set(pW0[0:8]).at[8:10].set(-pW0[16:18])
    k0s = jnp.kron(i8, ws)
    k0d = jnp.kron(i8, wd)
    b0 = jnp.tile(pb0, 8).reshape(1, 256)
    k1 = jnp.kron(i8, pW1)
    b1 = jnp.tile(pb1, 8).reshape(1, 256)
    k2 = jnp.kron(i8, pW2)
    b2 = jnp.tile(pb2, 8).reshape(1, 8)

    sp = srows.reshape(EPAD // 8, 128)
    dp = drows.reshape(EPAD // 8, 128)
    m2d = _run_phi(sp, dp, k0s, k0d, b0, k1, b1, k2, b2)

    mrows = m2d.reshape(EPAD // 128, 128)
    agg2 = _sc_scatter(didx, mrows)

    i16 = jnp.eye(16, dtype=f32)
    g0x = jnp.kron(i16, gW0[0:8])
    g0a = jnp.kron(i16, gW0[8:9])
    gb0t = jnp.tile(gb0, 16).reshape(1, 512)
    g1 = jnp.kron(i16, gW1)
    gb1t = jnp.tile(gb1, 16).reshape(1, 512)
    g2 = jnp.kron(i16, gW2)
    gb2t = jnp.tile(gb2, 16).reshape(1, 16)

    xp = x.reshape(N // 16, 128)
    a0 = agg2[0, :N].reshape(N // 16, 16)
    a1 = agg2[1, :N].reshape(N // 16, 16)
    xl = x[:, 7].reshape(N // 16, 16)

    out = _run_gamma(xp, a0, a1, xl, g0x, g0a, gb0t, g1, gb1t, g2, gb2t)
    return out.reshape(N, 1)


# speedup vs baseline: 1.8391x; 1.8125x over previous
import functools

import jax
import jax.numpy as jnp
from jax import lax
from jax.experimental import pallas as pl
from jax.experimental.pallas import tpu as pltpu
from jax.experimental.pallas import tpu_sc as plsc

N = 100000
E = 3200000
NC = 2
NS = 16
NW = NC * NS
EPAD = 3276800
GCHUNK = 1024
GITER = EPAD // NW // GCHUNK
GC_SLOW = 100
GC_FAST = 100
SC_SLOW = 100
SC_FAST = 100
NAGG = 100352

_mesh = plsc.VectorSubcoreMesh(core_axis_name="c", subcore_axis_name="s")
_sc_params = pltpu.CompilerParams(use_tc_tiling_on_sc=False)


@functools.partial(
    pl.kernel,
    out_type=(
        jax.ShapeDtypeStruct((EPAD, 8), jnp.uint32),
        jax.ShapeDtypeStruct((EPAD, 8), jnp.uint32),
    ),
    mesh=_mesh,
    scratch_types=[
        pltpu.VMEM((2, 8, 128), jnp.int32),
        pltpu.VMEM((2, 8, 128), jnp.int32),
        pltpu.VMEM((2, GCHUNK, 8), jnp.uint32),
        pltpu.VMEM((2, GCHUNK, 8), jnp.uint32),
        pltpu.VMEM_SHARED((N, 8), jnp.uint32),
        pltpu.SemaphoreType.DMA,
        pltpu.SemaphoreType.DMA,
        pltpu.SemaphoreType.DMA,
    ],
    compiler_params=_sc_params,
)
def _sc_gather(table_hbm, sidx_hbm, didx_hbm, srows_hbm, drows_hbm,
               idxs_v, idxd_v, bufs_v, bufd_v, table_sp, gsem, wsem0, wsem1):
    c = lax.axis_index("c")
    s = lax.axis_index("s")
    wsems = (wsem0, wsem1)
    pert = jnp.where(c == 1, GC_SLOW, GC_FAST)
    start = jnp.where(c == 1, s * GC_SLOW, NS * GC_SLOW + s * GC_FAST)

    @pl.when(s < 4)
    def _stage():
        off = pl.multiple_of(s * (N // 4), 8)
        pltpu.sync_copy(table_hbm.at[pl.ds(off, N // 4)],
                        table_sp.at[pl.ds(off, N // 4)])

    plsc.subcore_barrier()

    def outer(g, carry):
        for p in range(2):
            i = g * 2 + p
            base = pl.multiple_of((start + i) * GCHUNK, 1024)
            row0 = pl.multiple_of(base // 128, 8)

            @pl.when(g > 0)
            def _drain():
                pltpu.make_async_copy(
                    bufs_v.at[p], srows_hbm.at[pl.ds(base, GCHUNK)],
                    wsems[p]).wait()
                pltpu.make_async_copy(
                    bufd_v.at[p], drows_hbm.at[pl.ds(base, GCHUNK)],
                    wsems[p]).wait()

            pltpu.sync_copy(sidx_hbm.at[pl.ds(row0, 8)], idxs_v.at[p])
            pltpu.sync_copy(didx_hbm.at[pl.ds(row0, 8)], idxd_v.at[p])
            descs = []
            for j in range(8):
                descs.append(pltpu.async_copy(
                    table_sp.at[idxs_v.at[p].at[j]],
                    bufs_v.at[p].at[pl.ds(j * 128, 128)], gsem))
                descs.append(pltpu.async_copy(
                    table_sp.at[idxd_v.at[p].at[j]],
                    bufd_v.at[p].at[pl.ds(j * 128, 128)], gsem))
            for d in descs:
                d.wait()
            pltpu.async_copy(
                bufs_v.at[p], srows_hbm.at[pl.ds(base, GCHUNK)], wsems[p])
            pltpu.async_copy(
                bufd_v.at[p], drows_hbm.at[pl.ds(base, GCHUNK)], wsems[p])
        return carry

    lax.fori_loop(0, pert // 2, outer, 0)

    for p in range(2):
        base = pl.multiple_of((start + pert - 2 + p) * GCHUNK, 1024)
        pltpu.make_async_copy(
            bufs_v.at[p], srows_hbm.at[pl.ds(base, GCHUNK)], wsems[p]).wait()
        pltpu.make_async_copy(
            bufd_v.at[p], drows_hbm.at[pl.ds(base, GCHUNK)], wsems[p]).wait()


@functools.partial(
    pl.kernel,
    out_type=jax.ShapeDtypeStruct((NC, NAGG), jnp.float32),
    mesh=_mesh,
    scratch_types=[
        pltpu.VMEM((8, 128), jnp.int32),
        pltpu.VMEM((8, 128), jnp.float32),
        pltpu.VMEM((2048,), jnp.float32),
        pltpu.VMEM_SHARED((NAGG,), jnp.float32),
        pltpu.SemaphoreType.DMA,
    ],
    compiler_params=_sc_params,
)
def _sc_scatter(didx_hbm, m_hbm, agg_hbm, idx_v, val_v, zbuf_v, agg_sp, sem):
    c = lax.axis_index("c")
    s = lax.axis_index("s")

    @pl.when(s == 0)
    def _zero():
        def zb(k, carry):
            zbuf_v[pl.ds(k * 16, 16)] = jnp.zeros((16,), jnp.float32)
            return carry
        lax.fori_loop(0, 2048 // 16, zb, 0)

        def zs(k, carry):
            pltpu.sync_copy(zbuf_v, agg_sp.at[pl.ds(k * 2048, 2048)])
            return carry
        lax.fori_loop(0, NAGG // 2048, zs, 0)

    plsc.subcore_barrier()

    pert = jnp.where(c == 1, SC_SLOW, SC_FAST)
    start = jnp.where(c == 1, s * SC_SLOW, NS * SC_SLOW + s * SC_FAST)

    def chunk(i, carry):
        row0 = pl.multiple_of((start + i) * 8, 8)
        pltpu.sync_copy(didx_hbm.at[pl.ds(row0, 8)], idx_v)
        pltpu.sync_copy(m_hbm.at[pl.ds(row0, 8)], val_v)
        descs = []
        for j in range(8):
            descs.append(pltpu.async_copy(
                val_v.at[j], agg_sp.at[idx_v.at[j]], sem, add=True))
        for d in descs:
            d.wait()
        return carry

    lax.fori_loop(0, pert, chunk, 0)

    plsc.subcore_barrier()

    @pl.when(s == 0)
    def _writeout():
        pltpu.sync_copy(agg_sp, agg_hbm.at[c])


def _planes(r):
    lo = lax.bitcast_convert_type(
        (r & jnp.uint32(0xFFFF)).astype(jnp.uint16), jnp.bfloat16)
    hi = lax.bitcast_convert_type(
        (r >> 16).astype(jnp.uint16), jnp.bfloat16)
    return lo, hi


def _phi_body(s_ref, d_ref, k0_ref, b0_ref, k1_ref, b1_ref,
              k2_ref, b2_ref, m_ref):
    bf = jnp.bfloat16
    lo_s, hi_s = _planes(s_ref[...])
    lo_d, hi_d = _planes(d_ref[...])
    v = jnp.concatenate([lo_s, hi_s, lo_d, hi_d], axis=1)
    h = jnp.tanh(
        jnp.dot(v, k0_ref[...], preferred_element_type=jnp.float32)
        + b0_ref[...])
    h = jnp.tanh(
        jnp.dot(h.astype(bf), k1_ref[...], preferred_element_type=jnp.float32)
        + b1_ref[...])
    m_ref[...] = (
        jnp.dot(h.astype(bf), k2_ref[...], preferred_element_type=jnp.float32)
        + b2_ref[...])


def _run_phi(sp, dp, k0, b0, k1, b1, k2, b2):
    rows = EPAD // 16
    blk = 4096
    grid = rows // blk
    full = lambda shape: pl.BlockSpec(shape, lambda i: (0, 0))
    return pl.pallas_call(
        _phi_body,
        grid=(grid,),
        in_specs=[
            pl.BlockSpec((blk, 128), lambda i: (i, 0)),
            pl.BlockSpec((blk, 128), lambda i: (i, 0)),
            full((512, 512)), full((1, 512)),
            full((512, 512)), full((1, 512)),
            full((512, 16)), full((1, 16)),
        ],
        out_specs=pl.BlockSpec((blk, 16), lambda i: (i, 0)),
        out_shape=jax.ShapeDtypeStruct((rows, 16), jnp.float32),
    )(sp, dp, k0, b0, k1, b1, k2, b2)


def _gamma_body(x_ref, a0_ref, a1_ref, xl_ref, g0x_ref, g0a_ref, b0_ref,
                g1_ref, b1_ref, g2_ref, b2_ref, o_ref):
    a = a0_ref[...] + a1_ref[...]
    h = jnp.tanh(
        jnp.dot(x_ref[...], g0x_ref[...], preferred_element_type=jnp.float32)
        + jnp.dot(a, g0a_ref[...], preferred_element_type=jnp.float32)
        + b0_ref[...])
    h = jnp.tanh(
        jnp.dot(h, g1_ref[...], preferred_element_type=jnp.float32)
        + b1_ref[...])
    o_ref[...] = (xl_ref[...]
                  + jnp.dot(h, g2_ref[...], preferred_element_type=jnp.float32)
                  + b2_ref[...])


def _run_gamma(xp, a0, a1, xl, g0x, g0a, b0, g1, b1, g2, b2):
    rows = N // 16
    return pl.pallas_call(
        _gamma_body,
        out_shape=jax.ShapeDtypeStruct((rows, 16), jnp.float32),
    )(xp, a0, a1, xl, g0x, g0a, b0, g1, b1, g2, b2)


def kernel(x, pos, edge_index, pW0, pb0, pW1, pb1, pW2, pb2,
           gW0, gb0, gW1, gb1, gW2, gb2):
    f32 = jnp.float32
    u32 = jnp.uint32
    xu = lax.bitcast_convert_type(
        x.astype(jnp.bfloat16).reshape(N, 4, 2), u32)
    pu = lax.bitcast_convert_type(pos, u32) & jnp.uint32(0xFFFF0000)
    table = jnp.concatenate([xu, pu, jnp.zeros((N, 2), u32)], axis=1)

    src = edge_index[0]
    dst = edge_index[1]
    pad = EPAD - E
    src_p = jnp.concatenate([src, jnp.zeros((pad,), jnp.int32)])
    dst_p = jnp.concatenate([dst, jnp.full((pad,), N, jnp.int32)])
    sidx = src_p.reshape(EPAD // 128, 128)
    didx = dst_p.reshape(EPAD // 128, 128)

    srows, drows = _sc_gather(table, sidx, didx)

    bf = jnp.bfloat16
    i16 = jnp.eye(16, dtype=f32)
    ws_lo = (jnp.zeros((8, 32), f32).at[0].set(pW0[8]).at[1].set(pW0[10])
             .at[2].set(pW0[12]).at[3].set(pW0[14]))
    ws_hi = (jnp.zeros((8, 32), f32).at[0].set(pW0[9]).at[1].set(pW0[11])
             .at[2].set(pW0[13]).at[3].set(pW0[15])
             .at[4].set(pW0[16]).at[5].set(pW0[17]))
    wd_lo = (jnp.zeros((8, 32), f32).at[0].set(pW0[0]).at[1].set(pW0[2])
             .at[2].set(pW0[4]).at[3].set(pW0[6]))
    wd_hi = (jnp.zeros((8, 32), f32).at[0].set(pW0[1]).at[1].set(pW0[3])
             .at[2].set(pW0[5]).at[3].set(pW0[7])
             .at[4].set(-pW0[16]).at[5].set(-pW0[17]))
    k0 = jnp.concatenate(
        [jnp.kron(i16, ws_lo), jnp.kron(i16, ws_hi),
         jnp.kron(i16, wd_lo), jnp.kron(i16, wd_hi)], axis=0).astype(bf)
    b0 = jnp.tile(pb0, 16).reshape(1, 512)
    k1 = jnp.kron(i16, pW1).astype(bf)
    b1 = jnp.tile(pb1, 16).reshape(1, 512)
    k2 = jnp.kron(i16, pW2).astype(bf)
    b2 = jnp.tile(pb2, 16).reshape(1, 16)

    sp = srows.reshape(EPAD // 16, 128)
    dp = drows.reshape(EPAD // 16, 128)
    m2d = _run_phi(sp, dp, k0, b0, k1, b1, k2, b2)

    mrows = m2d.reshape(EPAD // 128, 128)
    agg2 = _sc_scatter(didx, mrows)

    i16 = jnp.eye(16, dtype=f32)
    g0x = jnp.kron(i16, gW0[0:8])
    g0a = jnp.kron(i16, gW0[8:9])
    gb0t = jnp.tile(gb0, 16).reshape(1, 512)
    g1 = jnp.kron(i16, gW1)
    gb1t = jnp.tile(gb1, 16).reshape(1, 512)
    g2 = jnp.kron(i16, gW2)
    gb2t = jnp.tile(gb2, 16).reshape(1, 16)

    xp = x.reshape(N // 16, 128)
    a0 = agg2[0, :N].reshape(N // 16, 16)
    a1 = agg2[1, :N].reshape(N // 16, 16)
    xl = x[:, 7].reshape(N // 16, 16)

    out = _run_gamma(xp, a0, a1, xl, g0x, g0a, gb0t, g1, gb1t, g2, gb2t)
    return out.reshape(N, 1)


# speedup vs baseline: 2.0606x; 1.1204x over previous
import functools

import jax
import jax.numpy as jnp
from jax import lax
from jax.experimental import pallas as pl
from jax.experimental.pallas import tpu as pltpu
from jax.experimental.pallas import tpu_sc as plsc

N = 100000
E = 3200000
NC = 2
NS = 16
NW = NC * NS
EPAD = 3276800
GCHUNK = 1024
GITER = EPAD // NW // GCHUNK
GC_SLOW = 100
GC_FAST = 100
SC_SLOW = 50
SC_FAST = 50
NAGG = 102400

_mesh = plsc.VectorSubcoreMesh(core_axis_name="c", subcore_axis_name="s")
_sc_params = pltpu.CompilerParams(use_tc_tiling_on_sc=False)


@functools.partial(
    pl.kernel,
    out_type=(
        jax.ShapeDtypeStruct((EPAD, 8), jnp.uint32),
        jax.ShapeDtypeStruct((EPAD, 8), jnp.uint32),
    ),
    mesh=_mesh,
    scratch_types=[
        pltpu.VMEM((2, 8, 128), jnp.int32),
        pltpu.VMEM((2, 8, 128), jnp.int32),
        pltpu.VMEM((2, GCHUNK, 8), jnp.uint32),
        pltpu.VMEM((2, GCHUNK, 8), jnp.uint32),
        pltpu.VMEM_SHARED((N, 8), jnp.uint32),
        pltpu.SemaphoreType.DMA,
        pltpu.SemaphoreType.DMA,
        pltpu.SemaphoreType.DMA,
    ],
    compiler_params=_sc_params,
)
def _sc_gather(table_hbm, sidx_hbm, didx_hbm, srows_hbm, drows_hbm,
               idxs_v, idxd_v, bufs_v, bufd_v, table_sp, gsem, wsem0, wsem1):
    c = lax.axis_index("c")
    s = lax.axis_index("s")
    wsems = (wsem0, wsem1)
    pert = jnp.where(c == 1, GC_SLOW, GC_FAST)
    start = jnp.where(c == 1, s * GC_SLOW, NS * GC_SLOW + s * GC_FAST)

    @pl.when(s < 4)
    def _stage():
        off = pl.multiple_of(s * (N // 4), 8)
        pltpu.sync_copy(table_hbm.at[pl.ds(off, N // 4)],
                        table_sp.at[pl.ds(off, N // 4)])

    plsc.subcore_barrier()

    def outer(g, carry):
        for p in range(2):
            i = g * 2 + p
            base = pl.multiple_of((start + i) * GCHUNK, 1024)
            row0 = pl.multiple_of(base // 128, 8)

            @pl.when(g > 0)
            def _drain():
                pltpu.make_async_copy(
                    bufs_v.at[p], srows_hbm.at[pl.ds(base, GCHUNK)],
                    wsems[p]).wait()
                pltpu.make_async_copy(
                    bufd_v.at[p], drows_hbm.at[pl.ds(base, GCHUNK)],
                    wsems[p]).wait()

            pltpu.sync_copy(sidx_hbm.at[pl.ds(row0, 8)], idxs_v.at[p])
            pltpu.sync_copy(didx_hbm.at[pl.ds(row0, 8)], idxd_v.at[p])
            descs = []
            for j in range(8):
                descs.append(pltpu.async_copy(
                    table_sp.at[idxs_v.at[p].at[j]],
                    bufs_v.at[p].at[pl.ds(j * 128, 128)], gsem))
                descs.append(pltpu.async_copy(
                    table_sp.at[idxd_v.at[p].at[j]],
                    bufd_v.at[p].at[pl.ds(j * 128, 128)], gsem))
            for d in descs:
                d.wait()
            pltpu.async_copy(
                bufs_v.at[p], srows_hbm.at[pl.ds(base, GCHUNK)], wsems[p])
            pltpu.async_copy(
                bufd_v.at[p], drows_hbm.at[pl.ds(base, GCHUNK)], wsems[p])
        return carry

    lax.fori_loop(0, pert // 2, outer, 0)

    for p in range(2):
        base = pl.multiple_of((start + pert - 2 + p) * GCHUNK, 1024)
        pltpu.make_async_copy(
            bufs_v.at[p], srows_hbm.at[pl.ds(base, GCHUNK)], wsems[p]).wait()
        pltpu.make_async_copy(
            bufd_v.at[p], drows_hbm.at[pl.ds(base, GCHUNK)], wsems[p]).wait()


@functools.partial(
    pl.kernel,
    out_type=jax.ShapeDtypeStruct((NC, N), jnp.float32),
    mesh=_mesh,
    scratch_types=[
        pltpu.VMEM((2, 16, 128), jnp.int32),
        pltpu.VMEM((2, 16, 128), jnp.float32),
        pltpu.VMEM((2048,), jnp.float32),
        pltpu.VMEM_SHARED((NAGG,), jnp.float32),
        pltpu.SemaphoreType.DMA,
        pltpu.SemaphoreType.DMA,
    ],
    compiler_params=_sc_params,
)
def _sc_scatter(didx_hbm, m_hbm, agg_hbm, idx_v, val_v, zbuf_v, agg_sp,
                lsem, ssem):
    c = lax.axis_index("c")
    s = lax.axis_index("s")

    def zb(k, carry):
        zbuf_v[pl.ds(k * 16, 16)] = jnp.zeros((16,), jnp.float32)
        return carry
    lax.fori_loop(0, 2048 // 16, zb, 0)

    nz = jnp.where(s < (NAGG // 2048) % NS, NAGG // 2048 // NS + 1,
                   NAGG // 2048 // NS)

    def zs(k, carry):
        off = pl.multiple_of((k * NS + s) * 2048, 2048)
        pltpu.sync_copy(zbuf_v, agg_sp.at[pl.ds(off, 2048)])
        return carry
    lax.fori_loop(0, nz, zs, 0)

    plsc.subcore_barrier()

    pert = jnp.where(c == 1, SC_SLOW, SC_FAST)
    start = jnp.where(c == 1, s * SC_SLOW, NS * SC_SLOW + s * SC_FAST)

    def load(i, p):
        row0 = pl.multiple_of((start + i) * 16, 16)
        pltpu.async_copy(didx_hbm.at[pl.ds(row0, 16)], idx_v.at[p], lsem)
        pltpu.async_copy(m_hbm.at[pl.ds(row0, 16)], val_v.at[p], lsem)

    def wait_load(p):
        pltpu.make_async_copy(didx_hbm.at[pl.ds(0, 16)], idx_v.at[p],
                              lsem).wait()
        pltpu.make_async_copy(m_hbm.at[pl.ds(0, 16)], val_v.at[p],
                              lsem).wait()

    load(0, 0)

    def chunk(g, carry):
        for p in range(2):
            i = g * 2 + p

            @pl.when(i + 1 < pert)
            def _prefetch():
                load(i + 1, 1 - p)

            wait_load(p)
            descs = []
            for j in range(16):
                descs.append(pltpu.async_copy(
                    val_v.at[p].at[j], agg_sp.at[idx_v.at[p].at[j]],
                    ssem, add=True))
            for d in descs:
                d.wait()
        return carry

    lax.fori_loop(0, pert // 2, chunk, 0)

    plsc.subcore_barrier()

    @pl.when(s == 0)
    def _writeout():
        pltpu.sync_copy(agg_sp.at[pl.ds(0, N)], agg_hbm.at[c])


def _planes(r):
    lo = lax.bitcast_convert_type(
        (r & jnp.uint32(0xFFFF)).astype(jnp.uint16), jnp.bfloat16)
    hi = lax.bitcast_convert_type(
        (r >> 16).astype(jnp.uint16), jnp.bfloat16)
    return lo, hi


def _phi_body(s_ref, d_ref, k0_ref, b0_ref, k1_ref, b1_ref,
              k2_ref, b2_ref, m_ref):
    bf = jnp.bfloat16
    lo_s, hi_s = _planes(s_ref[...])
    lo_d, hi_d = _planes(d_ref[...])
    v = jnp.concatenate([lo_s, hi_s, lo_d, hi_d], axis=1)
    h = jnp.tanh(
        jnp.dot(v, k0_ref[...], preferred_element_type=jnp.float32)
        + b0_ref[...])
    h = jnp.tanh(
        jnp.dot(h.astype(bf), k1_ref[...], preferred_element_type=jnp.float32)
        + b1_ref[...])
    m_ref[...] = (
        jnp.dot(h.astype(bf), k2_ref[...], preferred_element_type=jnp.float32)
        + b2_ref[...])


def _run_phi(sp, dp, k0, b0, k1, b1, k2, b2):
    rows = EPAD // 16
    blk = 4096
    grid = rows // blk
    full = lambda shape: pl.BlockSpec(shape, lambda i: (0, 0))
    return pl.pallas_call(
        _phi_body,
        grid=(grid,),
        in_specs=[
            pl.BlockSpec((blk, 128), lambda i: (i, 0)),
            pl.BlockSpec((blk, 128), lambda i: (i, 0)),
            full((512, 512)), full((1, 512)),
            full((512, 512)), full((1, 512)),
            full((512, 16)), full((1, 16)),
        ],
        out_specs=pl.BlockSpec((blk, 16), lambda i: (i, 0)),
        out_shape=jax.ShapeDtypeStruct((rows, 16), jnp.float32),
    )(sp, dp, k0, b0, k1, b1, k2, b2)


def _gamma_body(x_ref, a0_ref, a1_ref, xl_ref, g0x_ref, g0a_ref, b0_ref,
                g1_ref, b1_ref, g2_ref, b2_ref, o_ref):
    a = a0_ref[...] + a1_ref[...]
    h = jnp.tanh(
        jnp.dot(x_ref[...], g0x_ref[...], preferred_element_type=jnp.float32)
        + jnp.dot(a, g0a_ref[...], preferred_element_type=jnp.float32)
        + b0_ref[...])
    h = jnp.tanh(
        jnp.dot(h, g1_ref[...], preferred_element_type=jnp.float32)
        + b1_ref[...])
    o_ref[...] = (xl_ref[...]
                  + jnp.dot(h, g2_ref[...], preferred_element_type=jnp.float32)
                  + b2_ref[...])


def _run_gamma(xp, a0, a1, xl, g0x, g0a, b0, g1, b1, g2, b2):
    rows = N // 16
    return pl.pallas_call(
        _gamma_body,
        out_shape=jax.ShapeDtypeStruct((rows, 16), jnp.float32),
    )(xp, a0, a1, xl, g0x, g0a, b0, g1, b1, g2, b2)


def kernel(x, pos, edge_index, pW0, pb0, pW1, pb1, pW2, pb2,
           gW0, gb0, gW1, gb1, gW2, gb2):
    f32 = jnp.float32
    u32 = jnp.uint32
    xu = lax.bitcast_convert_type(
        x.astype(jnp.bfloat16).reshape(N, 4, 2), u32)
    pu = lax.bitcast_convert_type(pos, u32) & jnp.uint32(0xFFFF0000)
    table = jnp.concatenate([xu, pu, jnp.zeros((N, 2), u32)], axis=1)

    src = edge_index[0]
    dst = edge_index[1]
    pad = EPAD - E
    src_p = jnp.concatenate([src, jnp.zeros((pad,), jnp.int32)])
    dst_p = jnp.concatenate([dst, jnp.full((pad,), N, jnp.int32)])
    sidx = src_p.reshape(EPAD // 128, 128)
    didx = dst_p.reshape(EPAD // 128, 128)

    srows, drows = _sc_gather(table, sidx, didx)

    bf = jnp.bfloat16
    i16 = jnp.eye(16, dtype=f32)
    ws_lo = (jnp.zeros((8, 32), f32).at[0].set(pW0[8]).at[1].set(pW0[10])
             .at[2].set(pW0[12]).at[3].set(pW0[14]))
    ws_hi = (jnp.zeros((8, 32), f32).at[0].set(pW0[9]).at[1].set(pW0[11])
             .at[2].set(pW0[13]).at[3].set(pW0[15])
             .at[4].set(pW0[16]).at[5].set(pW0[17]))
    wd_lo = (jnp.zeros((8, 32), f32).at[0].set(pW0[0]).at[1].set(pW0[2])
             .at[2].set(pW0[4]).at[3].set(pW0[6]))
    wd_hi = (jnp.zeros((8, 32), f32).at[0].set(pW0[1]).at[1].set(pW0[3])
             .at[2].set(pW0[5]).at[3].set(pW0[7])
             .at[4].set(-pW0[16]).at[5].set(-pW0[17]))
    k0 = jnp.concatenate(
        [jnp.kron(i16, ws_lo), jnp.kron(i16, ws_hi),
         jnp.kron(i16, wd_lo), jnp.kron(i16, wd_hi)], axis=0).astype(bf)
    b0 = jnp.tile(pb0, 16).reshape(1, 512)
    k1 = jnp.kron(i16, pW1).astype(bf)
    b1 = jnp.tile(pb1, 16).reshape(1, 512)
    k2 = jnp.kron(i16, pW2).astype(bf)
    b2 = jnp.tile(pb2, 16).reshape(1, 16)

    sp = srows.reshape(EPAD // 16, 128)
    dp = drows.reshape(EPAD // 16, 128)
    m2d = _run_phi(sp, dp, k0, b0, k1, b1, k2, b2)

    mrows = m2d.reshape(EPAD // 128, 128)
    agg2 = _sc_scatter(didx, mrows)

    i16 = jnp.eye(16, dtype=f32)
    g0x = jnp.kron(i16, gW0[0:8])
    g0a = jnp.kron(i16, gW0[8:9])
    gb0t = jnp.tile(gb0, 16).reshape(1, 512)
    g1 = jnp.kron(i16, gW1)
    gb1t = jnp.tile(gb1, 16).reshape(1, 512)
    g2 = jnp.kron(i16, gW2)
    gb2t = jnp.tile(gb2, 16).reshape(1, 16)

    xp = x.reshape(N // 16, 128)
    a0 = agg2[0].reshape(N // 16, 16)
    a1 = agg2[1].reshape(N // 16, 16)
    xl = x[:, 7].reshape(N // 16, 16)

    out = _run_gamma(xp, a0, a1, xl, g0x, g0a, gb0t, g1, gb1t, g2, gb2t)
    return out.reshape(N, 1)


# speedup vs baseline: 2.3823x; 1.1561x over previous
import functools

import jax
import jax.numpy as jnp
from jax import lax
from jax.experimental import pallas as pl
from jax.experimental.pallas import tpu as pltpu
from jax.experimental.pallas import tpu_sc as plsc

N = 100000
E = 3200000
NC = 2
NS = 16
NW = NC * NS
GCHUNK = 1024
ECH = E // GCHUNK
CHB = ECH // NW
CHR = ECH - NW * CHB
GPAIR = (CHB + 2) // 2
NAGG = 102400

_mesh = plsc.VectorSubcoreMesh(core_axis_name="c", subcore_axis_name="s")
_sc_params = pltpu.CompilerParams(use_tc_tiling_on_sc=False)


def _work(wid):
    start = CHB * wid + jnp.minimum(wid, CHR)
    nch = jnp.where(wid < CHR, CHB + 1, CHB)
    return start, nch


@functools.partial(
    pl.kernel,
    out_type=(
        jax.ShapeDtypeStruct((E, 8), jnp.uint32),
        jax.ShapeDtypeStruct((E, 8), jnp.uint32),
    ),
    mesh=_mesh,
    scratch_types=[
        pltpu.VMEM((2, GCHUNK), jnp.int32),
        pltpu.VMEM((2, GCHUNK), jnp.int32),
        pltpu.VMEM((2, GCHUNK, 8), jnp.uint32),
        pltpu.VMEM((2, GCHUNK, 8), jnp.uint32),
        pltpu.VMEM_SHARED((N, 8), jnp.uint32),
        pltpu.SemaphoreType.DMA,
        pltpu.SemaphoreType.DMA,
        pltpu.SemaphoreType.DMA,
    ],
    compiler_params=_sc_params,
)
def _sc_gather(table_hbm, ei_hbm, srows_hbm, drows_hbm,
               idxs_v, idxd_v, bufs_v, bufd_v, table_sp, gsem, wsem0, wsem1):
    c = lax.axis_index("c")
    s = lax.axis_index("s")
    wsems = (wsem0, wsem1)
    start, nch = _work(s * NC + c)

    @pl.when(s < 4)
    def _stage():
        off = pl.multiple_of(s * (N // 4), 8)
        pltpu.sync_copy(table_hbm.at[pl.ds(off, N // 4)],
                        table_sp.at[pl.ds(off, N // 4)])

    plsc.subcore_barrier()

    def outer(g, carry):
        for p in range(2):
            i = g * 2 + p

            @pl.when(i < nch)
            def _half():
                base = pl.multiple_of((start + i) * GCHUNK, 1024)

                @pl.when(g > 0)
                def _drain():
                    pltpu.make_async_copy(
                        bufs_v.at[p], srows_hbm.at[pl.ds(0, GCHUNK)],
                        wsems[p]).wait()
                    pltpu.make_async_copy(
                        bufd_v.at[p], drows_hbm.at[pl.ds(0, GCHUNK)],
                        wsems[p]).wait()

                pltpu.sync_copy(ei_hbm.at[0, pl.ds(base, GCHUNK)],
                                idxs_v.at[p])
                pltpu.sync_copy(ei_hbm.at[1, pl.ds(base, GCHUNK)],
                                idxd_v.at[p])
                descs = []
                for j in range(8):
                    descs.append(pltpu.async_copy(
                        table_sp.at[idxs_v.at[p].at[pl.ds(j * 128, 128)]],
                        bufs_v.at[p].at[pl.ds(j * 128, 128)], gsem))
                    descs.append(pltpu.async_copy(
                        table_sp.at[idxd_v.at[p].at[pl.ds(j * 128, 128)]],
                        bufd_v.at[p].at[pl.ds(j * 128, 128)], gsem))
                for d in descs:
                    d.wait()
                pltpu.async_copy(
                    bufs_v.at[p], srows_hbm.at[pl.ds(base, GCHUNK)], wsems[p])
                pltpu.async_copy(
                    bufd_v.at[p], drows_hbm.at[pl.ds(base, GCHUNK)], wsems[p])
        return carry

    lax.fori_loop(0, GPAIR, outer, 0)

    for p in range(2):
        pltpu.make_async_copy(
            bufs_v.at[p], srows_hbm.at[pl.ds(0, GCHUNK)], wsems[p]).wait()
        pltpu.make_async_copy(
            bufd_v.at[p], drows_hbm.at[pl.ds(0, GCHUNK)], wsems[p]).wait()


@functools.partial(
    pl.kernel,
    out_type=jax.ShapeDtypeStruct((NC, N), jnp.float32),
    mesh=_mesh,
    scratch_types=[
        pltpu.VMEM((2, 8, 128), jnp.int32),
        pltpu.VMEM((2, 8, 128), jnp.float32),
        pltpu.VMEM((2048,), jnp.float32),
        pltpu.VMEM_SHARED((NAGG,), jnp.float32),
        pltpu.SemaphoreType.DMA,
        pltpu.SemaphoreType.DMA,
    ],
    compiler_params=_sc_params,
)
def _sc_scatter(didx_hbm, m_hbm, agg_hbm, idx_v, val_v, zbuf_v, agg_sp,
                lsem, ssem):
    c = lax.axis_index("c")
    s = lax.axis_index("s")

    def zb(k, carry):
        zbuf_v[pl.ds(k * 16, 16)] = jnp.zeros((16,), jnp.float32)
        return carry
    lax.fori_loop(0, 2048 // 16, zb, 0)

    nz = jnp.where(s < (NAGG // 2048) % NS, NAGG // 2048 // NS + 1,
                   NAGG // 2048 // NS)

    def zs(k, carry):
        off = pl.multiple_of((k * NS + s) * 2048, 2048)
        pltpu.sync_copy(zbuf_v, agg_sp.at[pl.ds(off, 2048)])
        return carry
    lax.fori_loop(0, nz, zs, 0)

    plsc.subcore_barrier()

    start, nch = _work(c * NS + s)

    def load(i, p):
        row0 = pl.multiple_of((start + i) * 8, 8)
        pltpu.async_copy(didx_hbm.at[pl.ds(row0, 8)], idx_v.at[p], lsem)
        pltpu.async_copy(m_hbm.at[pl.ds(row0, 8)], val_v.at[p], lsem)

    def wait_load(p):
        pltpu.make_async_copy(didx_hbm.at[pl.ds(0, 8)], idx_v.at[p],
                              lsem).wait()
        pltpu.make_async_copy(m_hbm.at[pl.ds(0, 8)], val_v.at[p],
                              lsem).wait()

    load(0, 0)

    def chunk(g, carry):
        for p in range(2):
            i = g * 2 + p

            @pl.when(i < nch)
            def _half():
                @pl.when(i + 1 < nch)
                def _prefetch():
                    load(i + 1, 1 - p)

                wait_load(p)
                descs = []
                for j in range(8):
                    descs.append(pltpu.async_copy(
                        val_v.at[p].at[j], agg_sp.at[idx_v.at[p].at[j]],
                        ssem, add=True))
                for d in descs:
                    d.wait()
        return carry

    lax.fori_loop(0, GPAIR, chunk, 0)

    plsc.subcore_barrier()

    @pl.when(s == 0)
    def _writeout():
        pltpu.sync_copy(agg_sp.at[pl.ds(0, N)], agg_hbm.at[c])


def _planes(r):
    lo = lax.bitcast_convert_type(
        (r & jnp.uint32(0xFFFF)).astype(jnp.uint16), jnp.bfloat16)
    hi = lax.bitcast_convert_type(
        (r >> 16).astype(jnp.uint16), jnp.bfloat16)
    return lo, hi


def _phi_body(s_ref, d_ref, k0_ref, b0_ref, k1_ref, b1_ref,
              k2_ref, b2_ref, m_ref):
    bf = jnp.bfloat16
    lo_s, hi_s = _planes(s_ref[...])
    lo_d, hi_d = _planes(d_ref[...])
    v = jnp.concatenate([lo_s, hi_s, lo_d, hi_d], axis=1)
    h = jnp.tanh(
        jnp.dot(v, k0_ref[...], preferred_element_type=jnp.float32)
        + b0_ref[...])
    h = jnp.tanh(
        jnp.dot(h.astype(bf), k1_ref[...], preferred_element_type=jnp.float32)
        + b1_ref[...])
    m_ref[...] = (
        jnp.dot(h.astype(bf), k2_ref[...], preferred_element_type=jnp.float32)
        + b2_ref[...])


def _run_phi(sp, dp, k0, b0, k1, b1, k2, b2):
    rows = E // 16
    blk = 4000
    grid = rows // blk
    full = lambda shape: pl.BlockSpec(shape, lambda i: (0, 0))
    return pl.pallas_call(
        _phi_body,
        grid=(grid,),
        in_specs=[
            pl.BlockSpec((blk, 128), lambda i: (i, 0)),
            pl.BlockSpec((blk, 128), lambda i: (i, 0)),
            full((512, 512)), full((1, 512)),
            full((512, 512)), full((1, 512)),
            full((512, 16)), full((1, 16)),
        ],
        out_specs=pl.BlockSpec((blk, 16), lambda i: (i, 0)),
        out_shape=jax.ShapeDtypeStruct((rows, 16), jnp.float32),
    )(sp, dp, k0, b0, k1, b1, k2, b2)


def _gamma_body(x_ref, a0_ref, a1_ref, xl_ref, g0x_ref, g0a_ref, b0_ref,
                g1_ref, b1_ref, g2_ref, b2_ref, o_ref):
    a = a0_ref[...] + a1_ref[...]
    h = jnp.tanh(
        jnp.dot(x_ref[...], g0x_ref[...], preferred_element_type=jnp.float32)
        + jnp.dot(a, g0a_ref[...], preferred_element_type=jnp.float32)
        + b0_ref[...])
    h = jnp.tanh(
        jnp.dot(h, g1_ref[...], preferred_element_type=jnp.float32)
        + b1_ref[...])
    o_ref[...] = (xl_ref[...]
                  + jnp.dot(h, g2_ref[...], preferred_element_type=jnp.float32)
                  + b2_ref[...])


def _run_gamma(xp, a0, a1, xl, g0x, g0a, b0, g1, b1, g2, b2):
    rows = N // 16
    return pl.pallas_call(
        _gamma_body,
        out_shape=jax.ShapeDtypeStruct((rows, 16), jnp.float32),
    )(xp, a0, a1, xl, g0x, g0a, b0, g1, b1, g2, b2)


def kernel(x, pos, edge_index, pW0, pb0, pW1, pb1, pW2, pb2,
           gW0, gb0, gW1, gb1, gW2, gb2):
    f32 = jnp.float32
    u32 = jnp.uint32
    xu = lax.bitcast_convert_type(
        x.astype(jnp.bfloat16).reshape(N, 4, 2), u32)
    pu = lax.bitcast_convert_type(pos, u32) & jnp.uint32(0xFFFF0000)
    table = jnp.concatenate([xu, pu, jnp.zeros((N, 2), u32)], axis=1)

    srows, drows = _sc_gather(table, edge_index)

    bf = jnp.bfloat16
    i16 = jnp.eye(16, dtype=f32)
    ws_lo = (jnp.zeros((8, 32), f32).at[0].set(pW0[8]).at[1].set(pW0[10])
             .at[2].set(pW0[12]).at[3].set(pW0[14]))
    ws_hi = (jnp.zeros((8, 32), f32).at[0].set(pW0[9]).at[1].set(pW0[11])
             .at[2].set(pW0[13]).at[3].set(pW0[15])
             .at[4].set(pW0[16]).at[5].set(pW0[17]))
    wd_lo = (jnp.zeros((8, 32), f32).at[0].set(pW0[0]).at[1].set(pW0[2])
             .at[2].set(pW0[4]).at[3].set(pW0[6]))
    wd_hi = (jnp.zeros((8, 32), f32).at[0].set(pW0[1]).at[1].set(pW0[3])
             .at[2].set(pW0[5]).at[3].set(pW0[7])
             .at[4].set(-pW0[16]).at[5].set(-pW0[17]))
    k0 = jnp.concatenate(
        [jnp.kron(i16, ws_lo), jnp.kron(i16, ws_hi),
         jnp.kron(i16, wd_lo), jnp.kron(i16, wd_hi)], axis=0).astype(bf)
    b0 = jnp.tile(pb0, 16).reshape(1, 512)
    k1 = jnp.kron(i16, pW1).astype(bf)
    b1 = jnp.tile(pb1, 16).reshape(1, 512)
    k2 = jnp.kron(i16, pW2).astype(bf)
    b2 = jnp.tile(pb2, 16).reshape(1, 16)

    sp = srows.reshape(E // 16, 128)
    dp = drows.reshape(E // 16, 128)
    m2d = _run_phi(sp, dp, k0, b0, k1, b1, k2, b2)

    didx = edge_index[1].reshape(E // 128, 128)
    mrows = m2d.reshape(E // 128, 128)
    agg2 = _sc_scatter(didx, mrows)

    g0x = jnp.kron(i16, gW0[0:8])
    g0a = jnp.kron(i16, gW0[8:9])
    gb0t = jnp.tile(gb0, 16).reshape(1, 512)
    g1 = jnp.kron(i16, gW1)
    gb1t = jnp.tile(gb1, 16).reshape(1, 512)
    g2 = jnp.kron(i16, gW2)
    gb2t = jnp.tile(gb2, 16).reshape(1, 16)

    xp = x.reshape(N // 16, 128)
    a0 = agg2[0].reshape(N // 16, 16)
    a1 = agg2[1].reshape(N // 16, 16)
    xl = x[:, 7].reshape(N // 16, 16)

    out = _run_gamma(xp, a0, a1, xl, g0x, g0a, gb0t, g1, gb1t, g2, gb2t)
    return out.reshape(N, 1)
